# baseline (speedup 1.0000x reference)
import jax
import jax.numpy as jnp
from jax import lax
from jax.experimental import pallas as pl
from jax.experimental.pallas import tpu as pltpu
from jax.experimental.pallas import tpu_sc as plsc

HIST = 50
D = 128
B = 4096
NC, NS = 2, 16
NW = NC * NS
RW = B // NW
K = 8
NCHUNK = RW // K
NVEC = D // 16


def _sc_body(obs_hbm, buf_hbm, in_v, out_v, sem0, sem1):
    c = lax.axis_index("c")
    s = lax.axis_index("s")
    wid = s * NC + c
    base = wid * RW

    lane = lax.iota(jnp.int32, 16)
    zm = (lane < 6) | ((lane >= 9) & (lane < 12))
    zero = jnp.zeros((16,), jnp.float32)
    sems = (sem0, sem1)

    def chunk_pair(c0):
        for b in range(2):
            ch = c0 + b
            row0 = base + ch * K

            @pl.when(ch >= 2)
            def _wait():
                pltpu.make_async_copy(
                    out_v.at[b], buf_hbm.at[pl.ds(row0, K)], sems[b]).wait()

            pltpu.sync_copy(obs_hbm.at[pl.ds(row0, K)], in_v)
            for r in range(K):
                vecs = [in_v[r, pl.ds(jv * 16, 16)] for jv in range(NVEC)]
                v0m = jnp.where(zm, zero, vecs[0])

                def slot_body(sl, _v0m=v0m, _vecs=vecs, _r=r, _b=b):
                    out_v[_b, _r, sl, pl.ds(0, 16)] = _v0m
                    for jv in range(1, NVEC):
                        out_v[_b, _r, sl, pl.ds(jv * 16, 16)] = _vecs[jv]

                pl.loop(0, HIST)(slot_body)
                out_v[b, r, HIST - 1, pl.ds(0, 16)] = vecs[0]
            pltpu.async_copy(out_v.at[b], buf_hbm.at[pl.ds(row0, K)], sems[b])

    pl.loop(0, NCHUNK, step=2)(chunk_pair)
    for b in range(2):
        pltpu.make_async_copy(
            out_v.at[b], buf_hbm.at[pl.ds(base, K)], sems[b]).wait()


def _mask_body(mask_ref):
    mask_ref[...] = lax.broadcasted_iota(
        jnp.int32, (B, HIST), 1) < (HIST - 1)


def kernel(obs):
    if obs.ndim == 1:
        obs = obs[:, None]
    mesh = plsc.VectorSubcoreMesh(core_axis_name="c", subcore_axis_name="s")
    buf = pl.kernel(
        _sc_body,
        out_type=jax.ShapeDtypeStruct((B, HIST, D), jnp.float32),
        mesh=mesh,
        scratch_types=[
            pltpu.VMEM((K, D), jnp.float32),
            pltpu.VMEM((2, K, HIST, D), jnp.float32),
            pltpu.SemaphoreType.DMA,
            pltpu.SemaphoreType.DMA,
        ],
        compiler_params=pltpu.CompilerParams(use_tc_tiling_on_sc=True),
    )(obs)
    mask = pl.pallas_call(
        _mask_body,
        out_shape=jax.ShapeDtypeStruct((B, HIST), jnp.bool_),
    )()
    return buf, mask


# speedup vs baseline: 2.4404x; 2.4404x over previous
import jax
import jax.numpy as jnp
from jax import lax
from jax.experimental import pallas as pl
from jax.experimental.pallas import tpu as pltpu
from jax.experimental.pallas import tpu_sc as plsc

HIST = 50
D = 128
B = 4096
NC, NS = 2, 16
NW = NC * NS
RW = B // NW
NVEC = D // 16


def _sc_body(obs_hbm, buf_hbm, in_v, msk_v, sem):
    c = lax.axis_index("c")
    s = lax.axis_index("s")
    wid = s * NC + c
    base = wid * RW

    lane = lax.iota(jnp.int32, 16)
    zm = (lane < 6) | ((lane >= 9) & (lane < 12))
    zero = jnp.zeros((16,), jnp.float32)

    pltpu.sync_copy(obs_hbm.at[pl.ds(base, RW)], in_v)

    def fill_body(r):
        msk_v[r, pl.ds(0, 16)] = jnp.where(zm, zero, in_v[r, pl.ds(0, 16)])
        for jv in range(1, NVEC):
            msk_v[r, pl.ds(jv * 16, 16)] = in_v[r, pl.ds(jv * 16, 16)]

    pl.loop(0, RW, unroll=4)(fill_body)

    def issue_body(sl):
        pltpu.async_copy(msk_v, buf_hbm.at[sl, pl.ds(base, RW)], sem)

    pl.loop(0, HIST - 1)(issue_body)
    pltpu.async_copy(in_v, buf_hbm.at[HIST - 1, pl.ds(base, RW)], sem)

    def drain_body(sl):
        pltpu.make_async_copy(
            msk_v, buf_hbm.at[0, pl.ds(base, RW)], sem).wait()

    pl.loop(0, HIST)(drain_body)


def _mask_body(mask_ref):
    mask_ref[...] = lax.broadcasted_iota(
        jnp.int32, (HIST, B), 0) < (HIST - 1)


def kernel(obs):
    if obs.ndim == 1:
        obs = obs[:, None]
    mesh = plsc.VectorSubcoreMesh(core_axis_name="c", subcore_axis_name="s")
    buf_t = pl.kernel(
        _sc_body,
        out_type=jax.ShapeDtypeStruct((HIST, B, D), jnp.float32),
        mesh=mesh,
        scratch_types=[
            pltpu.VMEM((RW, D), jnp.float32),
            pltpu.VMEM((RW, D), jnp.float32),
            pltpu.SemaphoreType.DMA,
        ],
        compiler_params=pltpu.CompilerParams(use_tc_tiling_on_sc=True),
    )(obs)
    mask_t = pl.pallas_call(
        _mask_body,
        out_shape=jax.ShapeDtypeStruct((HIST, B), jnp.bool_),
    )()
    return jnp.transpose(buf_t, (1, 0, 2)), jnp.transpose(mask_t, (1, 0))


# speedup vs baseline: 2.4699x; 1.0121x over previous
import jax
import jax.numpy as jnp
from jax import lax
from jax.experimental import pallas as pl
from jax.experimental.pallas import tpu as pltpu
from jax.experimental.pallas import tpu_sc as plsc

HIST = 50
D = 128
B = 4096
NC, NS = 2, 16
NW = NC * NS
RW = B // NW
NVEC = D // 16


def _sc_body(obs_hbm, buf_hbm, in_v, msk_v, sem):
    c = lax.axis_index("c")
    s = lax.axis_index("s")
    wid = s * NC + c
    base = wid * RW

    lane = lax.iota(jnp.int32, 16)
    zm = (lane < 6) | ((lane >= 9) & (lane < 12))
    zero = jnp.zeros((16,), jnp.float32)

    pltpu.sync_copy(obs_hbm.at[pl.ds(base, RW)], in_v)
    pltpu.async_copy(in_v, buf_hbm.at[HIST - 1, pl.ds(base, RW)], sem)

    def fill_body(r):
        msk_v[r, pl.ds(0, 16)] = jnp.where(zm, zero, in_v[r, pl.ds(0, 16)])
        for jv in range(1, NVEC):
            msk_v[r, pl.ds(jv * 16, 16)] = in_v[r, pl.ds(jv * 16, 16)]

    pl.loop(0, RW, unroll=4)(fill_body)

    def issue_body(sl):
        pltpu.async_copy(msk_v, buf_hbm.at[sl, pl.ds(base, RW)], sem)

    pl.loop(0, HIST - 1)(issue_body)

    def drain_body(sl):
        pltpu.make_async_copy(
            msk_v, buf_hbm.at[0, pl.ds(base, RW)], sem).wait()

    pl.loop(0, HIST)(drain_body)


def _mask_body(mask_ref):
    mask_ref[...] = lax.broadcasted_iota(
        jnp.int32, (HIST, B), 0) < (HIST - 1)


def kernel(obs):
    if obs.ndim == 1:
        obs = obs[:, None]
    mesh = plsc.VectorSubcoreMesh(core_axis_name="c", subcore_axis_name="s")
    buf_t = pl.kernel(
        _sc_body,
        out_type=jax.ShapeDtypeStruct((HIST, B, D), jnp.float32),
        mesh=mesh,
        scratch_types=[
            pltpu.VMEM((RW, D), jnp.float32),
            pltpu.VMEM((RW, D), jnp.float32),
            pltpu.SemaphoreType.DMA,
        ],
        compiler_params=pltpu.CompilerParams(use_tc_tiling_on_sc=True),
    )(obs)
    mask_t = pl.pallas_call(
        _mask_body,
        out_shape=jax.ShapeDtypeStruct((HIST, B), jnp.bool_),
    )()
    return jnp.transpose(buf_t, (1, 0, 2)), jnp.transpose(mask_t, (1, 0))
